# baseline (speedup 1.0000x reference)
import functools

import jax
import jax.numpy as jnp
from jax import lax
from jax.experimental import pallas as pl
from jax.experimental.pallas import tpu as pltpu
from jax.experimental.pallas import tpu_sc as plsc

_CW = 512
_LOGW = 9
_NCB = 8


@functools.lru_cache(maxsize=None)
def _make_lookup(V, D, B):
    info = plsc.get_sparse_core_info()
    NC, NS = info.num_cores, info.num_subcores
    NW = NC * NS
    assert B % (8 * NW) == 0, (B, NW)
    lanes_pad = ((V + 127) // 128) * 128
    max_off = lanes_pad - _CW
    tiles = (V + 127) // 128
    tiles_per_w = (tiles + NW - 1) // NW
    rng = tiles_per_w * 128
    n_win = (rng + _CW - 1) // _CW
    mesh = plsc.VectorSubcoreMesh(core_axis_name="c", subcore_axis_name="s")

    @functools.partial(
        pl.kernel,
        mesh=mesh,
        compiler_params=pltpu.CompilerParams(needs_layout_passes=False),
        out_type=jax.ShapeDtypeStruct((B * D,), jnp.float32),
        scratch_types=[
            pltpu.VMEM((B,), jnp.int32),
            pltpu.VMEM((B,), jnp.int32),
            pltpu.VMEM((B,), jnp.int32),
            pltpu.VMEM((B,), jnp.int32),
            pltpu.VMEM((((n_win + 16) // 16) * 16,), jnp.int32),
            pltpu.VMEM((2, D, _CW), jnp.float32),
            pltpu.VMEM((_NCB, 16, D), jnp.float32),
            pltpu.SMEM((_NCB,), jnp.int32),
            pltpu.SMEM((((n_win + 16) // 16) * 16 + 1,), jnp.int32),
            [pltpu.SemaphoreType.DMA] * 2,
            [pltpu.SemaphoreType.DMA] * _NCB,
        ],
    )
    def k(tT_hbm, idx_hbm, out_hbm, idx_v, mi, mr, sr, wcnt, banks, colb,
          nsm, wb, bsems, csems):
        si = idx_v
        wid = lax.axis_index("s") * NC + lax.axis_index("c")
        lo = wid * rng
        hi = jnp.minimum(lo + rng, V)
        n_cnt = wcnt.shape[0]

        def fetch(b, bk):
            off = pl.multiple_of(jnp.minimum(lo + b * _CW, max_off), 128)
            pltpu.async_copy(
                tT_hbm.at[:, pl.ds(off, _CW)], banks.at[bk], bsems[bk]
            )

        def bdrain(bk):
            pltpu.make_async_copy(
                tT_hbm.at[:, pl.ds(0, _CW)], banks.at[bk], bsems[bk]
            ).wait()

        fetch(0, 0)
        fetch(1, 1)
        for sl in range(_NCB):
            nsm[sl] = 0

        pltpu.sync_copy(idx_hbm, idx_v)

        iota16 = lax.iota(jnp.int32, 16)
        zeros16 = jnp.zeros((16,), jnp.int32)

        @pl.loop(0, n_cnt // 16)
        def _(m):
            wcnt[pl.ds(m * 16, 16)] = zeros16

        def scan_body(m, moff):
            rv = idx_v[pl.ds(m * 16, 16)]
            iv = iota16 + m * 16
            mask = (rv >= lo) & (rv < hi)
            cs = plsc.cumsum(mask.astype(jnp.int32))
            pos = jnp.maximum(moff + cs - 1, zeros16)
            plsc.store_scatter(mr, [pos], rv, mask=mask)
            plsc.store_scatter(mi, [pos], iv, mask=mask)
            win = jnp.minimum(
                lax.shift_right_logical(jnp.maximum(rv - lo, zeros16), _LOGW),
                jnp.full((16,), n_cnt - 1, jnp.int32),
            )
            plsc.addupdate_scatter(
                wcnt, [win], jnp.full((16,), 1, jnp.int32), mask=mask
            )
            return moff + jnp.full((16,), cs[15], jnp.int32)

        moff = pl.loop(0, B // 16, init_carry=zeros16)(scan_body)
        mcount = moff[0]
        mc_splat = jnp.full((16,), mcount, jnp.int32)
        nmc = (mcount + 15) >> 4

        run = zeros16
        for m in range(n_cnt // 16):
            v = wcnt[pl.ds(m * 16, 16)]
            cs = plsc.cumsum(v)
            excl = run + cs - v
            for j in range(16):
                wb[m * 16 + j] = excl[j]
            run = run + jnp.full((16,), cs[15], jnp.int32)
        wb[n_win] = mcount

        def batch_body(b, vc):
            bk = lax.rem(b, 2)
            for sbk in range(2):
                @pl.when(bk == sbk)
                def _():
                    bdrain(sbk)
            off_b = jnp.minimum(lo + b * _CW, max_off)
            bk_splat = jnp.full((16,), bk, jnp.int32)
            start = wb[b]
            end = wb[b + 1]
            b_splat = jnp.full((16,), b, jnp.int32)

            def place(m2, base):
                rvm = mr[pl.ds(m2 * 16, 16)]
                ivm = mi[pl.ds(m2 * 16, 16)]
                valid = (iota16 + m2 * 16) < mc_splat
                win = lax.shift_right_logical(
                    jnp.maximum(rvm - lo, zeros16), _LOGW
                )
                mask = (win == b_splat) & valid
                cs = plsc.cumsum(mask.astype(jnp.int32))
                pos = jnp.maximum(base + cs - 1, zeros16)
                plsc.store_scatter(sr, [pos], rvm, mask=mask)
                plsc.store_scatter(si, [pos], ivm, mask=mask)
                return base + jnp.full((16,), cs[15], jnp.int32)

            pl.loop(0, nmc, init_carry=jnp.full((16,), start, jnp.int32))(
                place
            )

            start_splat = jnp.full((16,), start, jnp.int32)
            end_splat = jnp.full((16,), end, jnp.int32)

            def visit(m2, vc):
                gpos = iota16 + m2 * 16
                rvm = sr[pl.ds(m2 * 16, 16)]
                ivm = si[pl.ds(m2 * 16, 16)]
                bmask = (gpos >= start_splat) & (gpos < end_splat)
                np_ = plsc.all_reduce_population_count(bmask)
                active = np_[0] > 0
                bmask_i = bmask.astype(jnp.int32)
                sl_dyn = lax.rem(vc[0], _NCB)
                sl_splat = jnp.full((16,), sl_dyn, jnp.int32)

                @pl.when(active)
                def _():
                    lg = rvm - off_b
                    lcl = jnp.clip(lg, 0, _CW - 1)
                    for c in range(D):
                        c_splat = jnp.full((16,), c, jnp.int32)
                        vals = plsc.load_gather(
                            banks, [bk_splat, c_splat, lcl]
                        )
                        plsc.store_scatter(
                            colb, [sl_splat, iota16, c_splat], vals,
                            mask=bmask,
                        )
                    for sl in range(_NCB):
                        @pl.when(sl_dyn == sl)
                        def _():
                            nold = nsm[sl]

                            @pl.loop(0, nold)
                            def _(_i):
                                pltpu.make_async_copy(
                                    colb.at[sl, 0],
                                    out_hbm.at[pl.ds(0, D)],
                                    csems[sl],
                                ).wait()

                            for j in range(16):
                                @pl.when(bmask_i[j] > 0)
                                def _():
                                    pltpu.async_copy(
                                        colb.at[sl, j],
                                        out_hbm.at[pl.ds(ivm[j] * D, D)],
                                        csems[sl],
                                    )
                            nsm[sl] = np_[0]

                return vc + jnp.full(
                    (16,), active.astype(jnp.int32), jnp.int32
                )

            vc = pl.loop(start >> 4, (end + 15) >> 4, init_carry=vc)(visit)

            @pl.when(b + 2 < n_win)
            def _():
                for sbk in range(2):
                    @pl.when(bk == sbk)
                    def _():
                        fetch(b + 2, sbk)

            return vc

        pl.loop(0, n_win, init_carry=zeros16)(batch_body)

        for sl in range(_NCB):
            nold = nsm[sl]

            @pl.loop(0, nold)
            def _(_i):
                pltpu.make_async_copy(
                    colb.at[sl, 0], out_hbm.at[pl.ds(0, D)], csems[sl]
                ).wait()

    return k


def kernel(node_vecs, country_idx):
    V, D = node_vecs.shape
    B = country_idx.shape[0]
    idx = country_idx.reshape(B).astype(jnp.int32)
    flat = _make_lookup(V, D, B)(node_vecs.T, idx)
    return flat.reshape(B, D)


# speedup vs baseline: 1.0317x; 1.0317x over previous
import functools

import jax
import jax.numpy as jnp
from jax import lax
from jax.experimental import pallas as pl
from jax.experimental.pallas import tpu as pltpu
from jax.experimental.pallas import tpu_sc as plsc

_CW = 512
_LOGW = 9
_NCB = 6


@functools.lru_cache(maxsize=None)
def _make_lookup(V, D, B):
    info = plsc.get_sparse_core_info()
    NC, NS = info.num_cores, info.num_subcores
    NW = NC * NS
    assert B % (8 * NW) == 0, (B, NW)
    lanes_pad = ((V + 127) // 128) * 128
    max_off = lanes_pad - _CW
    tiles = (V + 127) // 128
    tiles_per_w = (tiles + NW - 1) // NW
    rng = tiles_per_w * 128
    n_win = (rng + _CW - 1) // _CW
    mesh = plsc.VectorSubcoreMesh(core_axis_name="c", subcore_axis_name="s")

    @functools.partial(
        pl.kernel,
        mesh=mesh,
        compiler_params=pltpu.CompilerParams(needs_layout_passes=False),
        out_type=jax.ShapeDtypeStruct((B * D,), jnp.float32),
        scratch_types=[
            pltpu.VMEM((B,), jnp.int32),
            pltpu.VMEM((B,), jnp.int32),
            pltpu.VMEM((B,), jnp.int32),
            pltpu.VMEM((B,), jnp.int32),
            pltpu.VMEM((((n_win + 16) // 16) * 16,), jnp.int32),
            pltpu.VMEM((3, D, _CW), jnp.float32),
            pltpu.VMEM((_NCB, 16, D), jnp.float32),
            pltpu.SMEM((_NCB,), jnp.int32),
            pltpu.SMEM((((n_win + 16) // 16) * 16 + 1,), jnp.int32),
            [pltpu.SemaphoreType.DMA] * 3,
            [pltpu.SemaphoreType.DMA] * _NCB,
        ],
    )
    def k(tT_hbm, idx_hbm, out_hbm, idx_v, mi, mr, sr, wcnt, banks, colb,
          nsm, wb, bsems, csems):
        si = idx_v
        wid = lax.axis_index("s") * NC + lax.axis_index("c")
        lo = wid * rng
        hi = jnp.minimum(lo + rng, V)
        n_cnt = wcnt.shape[0]

        def fetch(b, bk):
            off = pl.multiple_of(jnp.minimum(lo + b * _CW, max_off), 128)
            h = _CW // 2
            for q in range(2):
                pltpu.async_copy(
                    tT_hbm.at[:, pl.ds(off + q * h, h)],
                    banks.at[bk, :, pl.ds(q * h, h)],
                    bsems[bk],
                )

        def bdrain(bk):
            h = _CW // 2
            for q in range(2):
                pltpu.make_async_copy(
                    tT_hbm.at[:, pl.ds(0, h)],
                    banks.at[bk, :, pl.ds(q * h, h)],
                    bsems[bk],
                ).wait()

        fetch(0, 0)
        fetch(1, 1)
        fetch(2, 2)
        for sl in range(_NCB):
            nsm[sl] = 0

        pltpu.sync_copy(idx_hbm, idx_v)

        iota16 = lax.iota(jnp.int32, 16)
        zeros16 = jnp.zeros((16,), jnp.int32)

        @pl.loop(0, n_cnt // 16)
        def _(m):
            wcnt[pl.ds(m * 16, 16)] = zeros16

        def scan_body(m, moff):
            rv = idx_v[pl.ds(m * 16, 16)]
            iv = iota16 + m * 16
            mask = (rv >= lo) & (rv < hi)
            cs = plsc.cumsum(mask.astype(jnp.int32))
            pos = jnp.maximum(moff + cs - 1, zeros16)
            plsc.store_scatter(mr, [pos], rv, mask=mask)
            plsc.store_scatter(mi, [pos], iv, mask=mask)
            win = jnp.minimum(
                lax.shift_right_logical(jnp.maximum(rv - lo, zeros16), _LOGW),
                jnp.full((16,), n_cnt - 1, jnp.int32),
            )
            plsc.addupdate_scatter(
                wcnt, [win], jnp.full((16,), 1, jnp.int32), mask=mask
            )
            return moff + jnp.full((16,), cs[15], jnp.int32)

        moff = pl.loop(0, B // 16, init_carry=zeros16)(scan_body)
        mcount = moff[0]
        mc_splat = jnp.full((16,), mcount, jnp.int32)
        nmc = (mcount + 15) >> 4

        run = zeros16
        for m in range(n_cnt // 16):
            v = wcnt[pl.ds(m * 16, 16)]
            cs = plsc.cumsum(v)
            excl = run + cs - v
            for j in range(16):
                wb[m * 16 + j] = excl[j]
            run = run + jnp.full((16,), cs[15], jnp.int32)
        wb[n_win] = mcount

        def batch_body(b, vc):
            bk = lax.rem(b, 3)
            bk3 = lax.rem(b + 3, 3)
            for sbk in range(3):
                @pl.when(bk == sbk)
                def _():
                    bdrain(sbk)

            @pl.when(b + 3 < n_win)
            def _():
                for sbk in range(3):
                    @pl.when(bk3 == sbk)
                    def _():
                        fetch(b + 3, sbk)

            off_b = jnp.minimum(lo + b * _CW, max_off)
            bk_splat = jnp.full((16,), bk, jnp.int32)
            start = wb[b]
            end = wb[b + 1]
            b_splat = jnp.full((16,), b, jnp.int32)

            def place(m2, base):
                rvm = mr[pl.ds(m2 * 16, 16)]
                ivm = mi[pl.ds(m2 * 16, 16)]
                valid = (iota16 + m2 * 16) < mc_splat
                win = lax.shift_right_logical(
                    jnp.maximum(rvm - lo, zeros16), _LOGW
                )
                mask = (win == b_splat) & valid
                cs = plsc.cumsum(mask.astype(jnp.int32))
                pos = jnp.maximum(base + cs - 1, zeros16)
                plsc.store_scatter(sr, [pos], rvm, mask=mask)
                plsc.store_scatter(si, [pos], ivm, mask=mask)
                return base + jnp.full((16,), cs[15], jnp.int32)

            pl.loop(0, nmc, init_carry=jnp.full((16,), start, jnp.int32))(
                place
            )

            start_splat = jnp.full((16,), start, jnp.int32)
            end_splat = jnp.full((16,), end, jnp.int32)

            def visit(m2, vc):
                gpos = iota16 + m2 * 16
                rvm = sr[pl.ds(m2 * 16, 16)]
                ivm = si[pl.ds(m2 * 16, 16)]
                bmask = (gpos >= start_splat) & (gpos < end_splat)
                np_ = plsc.all_reduce_population_count(bmask)
                active = np_[0] > 0
                bmask_i = bmask.astype(jnp.int32)
                sl_dyn = lax.rem(vc[0], _NCB)
                sl_splat = jnp.full((16,), sl_dyn, jnp.int32)

                @pl.when(active)
                def _():
                    lg = rvm - off_b
                    lcl = jnp.clip(lg, 0, _CW - 1)
                    for c in range(D):
                        c_splat = jnp.full((16,), c, jnp.int32)
                        vals = plsc.load_gather(
                            banks, [bk_splat, c_splat, lcl]
                        )
                        plsc.store_scatter(
                            colb, [sl_splat, iota16, c_splat], vals,
                            mask=bmask,
                        )
                    for sl in range(_NCB):
                        @pl.when(sl_dyn == sl)
                        def _():
                            nold = nsm[sl]

                            @pl.loop(0, nold)
                            def _(_i):
                                pltpu.make_async_copy(
                                    colb.at[sl, 0],
                                    out_hbm.at[pl.ds(0, D)],
                                    csems[sl],
                                ).wait()

                            for j in range(16):
                                @pl.when(bmask_i[j] > 0)
                                def _():
                                    pltpu.async_copy(
                                        colb.at[sl, j],
                                        out_hbm.at[pl.ds(ivm[j] * D, D)],
                                        csems[sl],
                                    )
                            nsm[sl] = np_[0]

                return vc + jnp.full(
                    (16,), active.astype(jnp.int32), jnp.int32
                )

            vc = pl.loop(start >> 4, (end + 15) >> 4, init_carry=vc)(visit)

            return vc

        pl.loop(0, n_win, init_carry=zeros16)(batch_body)

        for sl in range(_NCB):
            nold = nsm[sl]

            @pl.loop(0, nold)
            def _(_i):
                pltpu.make_async_copy(
                    colb.at[sl, 0], out_hbm.at[pl.ds(0, D)], csems[sl]
                ).wait()

    return k


def kernel(node_vecs, country_idx):
    V, D = node_vecs.shape
    B = country_idx.shape[0]
    idx = country_idx.reshape(B).astype(jnp.int32)
    flat = _make_lookup(V, D, B)(node_vecs.T, idx)
    return flat.reshape(B, D)


# speedup vs baseline: 1.1001x; 1.0663x over previous
import functools

import jax
import jax.numpy as jnp
from jax import lax
from jax.experimental import pallas as pl
from jax.experimental.pallas import tpu as pltpu
from jax.experimental.pallas import tpu_sc as plsc

_NB = 16
_LANES = 128


@functools.lru_cache(maxsize=None)
def _make_lookup(V, D, B):
    info = plsc.get_sparse_core_info()
    NC, NS = info.num_cores, info.num_subcores
    NW = NC * NS
    assert B % (8 * NW) == 0, (B, NW)
    b_per_w = B // NW
    n_groups = b_per_w // _NB
    assert b_per_w % _NB == 0
    mesh = plsc.VectorSubcoreMesh(core_axis_name="c", subcore_axis_name="s")

    @functools.partial(
        pl.kernel,
        mesh=mesh,
        compiler_params=pltpu.CompilerParams(needs_layout_passes=False),
        out_type=jax.ShapeDtypeStruct((D, B), jnp.float32),
        scratch_types=[
            pltpu.VMEM((b_per_w,), jnp.int32),
            pltpu.VMEM((_NB, D, _LANES), jnp.float32),
            pltpu.VMEM((D, b_per_w), jnp.float32),
            [pltpu.SemaphoreType.DMA] * _NB,
        ],
    )
    def k(tT_hbm, idx_hbm, out_hbm, idx_v, tbuf, cols_v, sems):
        wid = lax.axis_index("s") * NC + lax.axis_index("c")
        base = wid * b_per_w
        pltpu.sync_copy(idx_hbm.at[pl.ds(base, b_per_w)], idx_v)

        c_lo = lax.iota(jnp.int32, 16)
        c_hi = c_lo + 16

        def fetch(r, slot):
            t_off = pl.multiple_of((r >> 7) << 7, _LANES)
            pltpu.async_copy(
                tT_hbm.at[:, pl.ds(t_off, _LANES)], tbuf.at[slot], sems[slot]
            )

        def drain(slot):
            pltpu.make_async_copy(
                tT_hbm.at[:, pl.ds(0, _LANES)], tbuf.at[slot], sems[slot]
            ).wait()

        def select(r, slot, j):
            l_splat = jnp.full((16,), r & 127, jnp.int32)
            j_splat = jnp.full((16,), j, jnp.int32)
            lo = plsc.load_gather(tbuf.at[slot], [c_lo, l_splat])
            hi = plsc.load_gather(tbuf.at[slot], [c_hi, l_splat])
            plsc.store_scatter(cols_v, [c_lo, j_splat], lo)
            plsc.store_scatter(cols_v, [c_hi, j_splat], hi)

        rv0 = idx_v[pl.ds(0, _NB)]
        for s in range(_NB):
            fetch(rv0[s], s)

        @pl.loop(0, n_groups)
        def group(g):
            j0 = g * _NB
            rv = idx_v[pl.ds(j0, _NB)]
            rv_next = idx_v[pl.ds(jnp.minimum(j0 + _NB, b_per_w - _NB), _NB)]
            for s in range(_NB):
                drain(s)
                select(rv[s], s, j0 + s)

                @pl.when(g + 1 < n_groups)
                def _():
                    fetch(rv_next[s], s)

        pltpu.sync_copy(cols_v, out_hbm.at[:, pl.ds(base, b_per_w)])

    return k


def kernel(node_vecs, country_idx):
    V, D = node_vecs.shape
    B = country_idx.shape[0]
    idx = country_idx.reshape(B).astype(jnp.int32)
    outT = _make_lookup(V, D, B)(node_vecs.T, idx)
    return outT.T
